# Initial kernel scaffold; baseline (speedup 1.0000x reference)
import jax
import jax.numpy as jnp
from jax.experimental import pallas as pl


def kernel(x, pos, pos_embeddings):
    raise NotImplementedError("write your pallas kernel here")


# speedup vs baseline: 4.4416x; 4.4416x over previous
import functools

import jax
import jax.numpy as jnp
from jax import lax
from jax.experimental import pallas as pl
from jax.experimental.pallas import tpu as pltpu
from jax.experimental.pallas import tpu_sc as plsc

_D = 256
_MAXPOS = 100
_NC, _NS = 2, 16
_NW = _NC * _NS
_CH = 128
_LANES = 16


def _sc_body(x_hbm, p0_hbm, p1_hbm, tab_hbm, out_hbm, p0_v, p1_v, idx_v,
             x_v, rows_v, sem):
    wid = lax.axis_index("s") * _NC + lax.axis_index("c")
    b_per_w = x_hbm.shape[0] // _NW
    n_chunks = b_per_w // _CH

    def chunk(c, carry):
        base = wid * b_per_w + c * _CH
        pltpu.sync_copy(p0_hbm.at[pl.ds(base, _CH)], p0_v)
        pltpu.sync_copy(p1_hbm.at[pl.ds(base, _CH)], p1_v)

        def mk_idx(j, carry2):
            s = j * _LANES
            idx_v[pl.ds(s, _LANES)] = (
                p0_v[pl.ds(s, _LANES)] * _MAXPOS + p1_v[pl.ds(s, _LANES)])
            return carry2

        lax.fori_loop(0, _CH // _LANES, mk_idx, 0, unroll=True)
        pltpu.sync_copy(x_hbm.at[pl.ds(base, _CH)], x_v)
        pltpu.async_copy(tab_hbm.at[idx_v], rows_v, sem).wait()

        def add_row(r, carry2):
            def add_vec(j, carry3):
                s = j * _LANES
                rows_v[r, pl.ds(s, _LANES)] = (
                    rows_v[r, pl.ds(s, _LANES)] + x_v[r, pl.ds(s, _LANES)])
                return carry3
            return lax.fori_loop(0, _D // _LANES, add_vec, carry2)

        lax.fori_loop(0, _CH, add_row, 0)
        pltpu.sync_copy(rows_v, out_hbm.at[pl.ds(base, _CH)])
        return carry

    lax.fori_loop(0, n_chunks, chunk, 0)


@jax.jit
def _run(x2, p0, p1, tab):
    B = x2.shape[0]
    mesh = plsc.VectorSubcoreMesh(core_axis_name="c", subcore_axis_name="s")
    k = pl.kernel(
        _sc_body,
        out_type=jax.ShapeDtypeStruct((B, _D), jnp.float32),
        mesh=mesh,
        scratch_types=[
            pltpu.VMEM((_CH,), jnp.int32),
            pltpu.VMEM((_CH,), jnp.int32),
            pltpu.VMEM((_CH,), jnp.int32),
            pltpu.VMEM((_CH, _D), jnp.float32),
            pltpu.VMEM((_CH, _D), jnp.float32),
            pltpu.SemaphoreType.DMA,
        ],
    )
    return k(x2, p0, p1, tab)


def kernel(x, pos, pos_embeddings):
    b, t, d = x.shape
    B = b * t
    x2 = x.reshape(B, d)
    p0 = pos[..., 0].reshape(B).astype(jnp.int32)
    p1 = pos[..., 1].reshape(B).astype(jnp.int32)
    tab = pos_embeddings.reshape(-1, d)
    return _run(x2, p0, p1, tab).reshape(b, t, d)


# speedup vs baseline: 12.7564x; 2.8720x over previous
import functools

import jax
import jax.numpy as jnp
from jax import lax
from jax.experimental import pallas as pl
from jax.experimental.pallas import tpu as pltpu
from jax.experimental.pallas import tpu_sc as plsc

_D = 256
_MAXPOS = 100
_NC, _NS = 2, 16
_NW = _NC * _NS
_CH = 32
_NBUF = 4
_LANES = 16


def _sc_body(x_hbm, p0_hbm, p1_hbm, tab_hbm, out_hbm, p0t, p1t, idx_all,
             xv, rv, in_sems, g_sems, o_sems):
    wid = lax.axis_index("s") * _NC + lax.axis_index("c")
    b_per_w = x_hbm.shape[0] // _NW
    n_chunks = b_per_w // _CH
    base_w = wid * b_per_w

    pltpu.sync_copy(p0_hbm.at[pl.ds(base_w, b_per_w)], p0t)
    pltpu.sync_copy(p1_hbm.at[pl.ds(base_w, b_per_w)], p1t)

    def mk_idx(c, carry):
        for u in range(_CH // _LANES):
            s = c * _CH + u * _LANES
            idx_all[c, pl.ds(u * _LANES, _LANES)] = (
                p0t[pl.ds(s, _LANES)] * _MAXPOS + p1t[pl.ds(s, _LANES)])
        return carry

    lax.fori_loop(0, n_chunks, mk_idx, 0)

    def issue_in(c, b):
        base = base_w + c * _CH
        pltpu.async_copy(x_hbm.at[pl.ds(base, _CH)], xv[b], in_sems[b])
        pltpu.async_copy(tab_hbm.at[idx_all.at[c]], rv[b], g_sems[b])

    for b in range(_NBUF - 1):
        issue_in(b, b)

    def group(g, carry):
        for b in range(_NBUF):
            c = g * _NBUF + b
            s3 = (b + _NBUF - 1) % _NBUF

            @pl.when(c + _NBUF - 1 < n_chunks)
            def _refill():
                @pl.when(c >= 1)
                def _drain():
                    pltpu.make_async_copy(
                        rv[s3], out_hbm.at[pl.ds(base_w, _CH)],
                        o_sems[s3]).wait()
                issue_in(c + _NBUF - 1, s3)

            pltpu.make_async_copy(
                x_hbm.at[pl.ds(base_w, _CH)], xv[b], in_sems[b]).wait()
            pltpu.make_async_copy(
                tab_hbm.at[idx_all.at[c]], rv[b], g_sems[b]).wait()

            def add_row(r, carry2):
                for u in range(_D // _LANES):
                    d = pl.ds(u * _LANES, _LANES)
                    rv[b][r, d] = rv[b][r, d] + xv[b][r, d]
                return carry2

            lax.fori_loop(0, _CH, add_row, 0)
            pltpu.async_copy(
                rv[b], out_hbm.at[pl.ds(base_w + c * _CH, _CH)], o_sems[b])
        return carry

    lax.fori_loop(0, n_chunks // _NBUF, group, 0)

    for b in range(_NBUF):
        pltpu.make_async_copy(
            rv[b], out_hbm.at[pl.ds(base_w, _CH)], o_sems[b]).wait()


@jax.jit
def _run(x2, p0, p1, tab):
    B = x2.shape[0]
    b_per_w = B // _NW
    n_chunks = b_per_w // _CH
    mesh = plsc.VectorSubcoreMesh(core_axis_name="c", subcore_axis_name="s")
    k = pl.kernel(
        _sc_body,
        out_type=jax.ShapeDtypeStruct((B, _D), jnp.float32),
        mesh=mesh,
        scratch_types=[
            pltpu.VMEM((b_per_w,), jnp.int32),
            pltpu.VMEM((b_per_w,), jnp.int32),
            pltpu.VMEM((n_chunks, _CH), jnp.int32),
            [pltpu.VMEM((_CH, _D), jnp.float32) for _ in range(_NBUF)],
            [pltpu.VMEM((_CH, _D), jnp.float32) for _ in range(_NBUF)],
            [pltpu.SemaphoreType.DMA for _ in range(_NBUF)],
            [pltpu.SemaphoreType.DMA for _ in range(_NBUF)],
            [pltpu.SemaphoreType.DMA for _ in range(_NBUF)],
        ],
    )
    return k(x2, p0, p1, tab)


def kernel(x, pos, pos_embeddings):
    b, t, d = x.shape
    B = b * t
    x2 = x.reshape(B, d)
    p0 = pos[..., 0].reshape(B).astype(jnp.int32)
    p1 = pos[..., 1].reshape(B).astype(jnp.int32)
    tab = pos_embeddings.reshape(-1, d)
    return _run(x2, p0, p1, tab).reshape(b, t, d)
